# Initial kernel scaffold; baseline (speedup 1.0000x reference)
import functools
import math

import jax
import jax.numpy as jnp
from jax.experimental import pallas as pl
from jax.experimental.pallas import tpu as pltpu

N, D, H, NE, K = 2048, 768, 768, 64, 8
TEMP_INV = 1.0 / math.e
NEG_INF = -1e30


def _router_body(x_ref, gw_ref, gb_ref, w_ref):
    scores = jax.lax.dot_general(
        x_ref[...], gw_ref[...], (((1,), (1,)), ((), ())),
        preferred_element_type=jnp.float32)
    scores = scores * TEMP_INV + gb_ref[...]
    m = jnp.max(scores, axis=1, keepdims=True)
    p = jnp.exp(scores - m)
    p = p / jnp.sum(p, axis=1, keepdims=True)

    e_iota = jax.lax.broadcasted_iota(jnp.int32, (N, NE), 1)
    work = p
    sel_total = jnp.zeros((N, NE), jnp.float32)
    for _ in range(K):
        mk = jnp.max(work, axis=1, keepdims=True)
        cand = jnp.where(work == mk, e_iota, NE)
        idx = jnp.min(cand, axis=1, keepdims=True)
        sel = (e_iota == idx).astype(jnp.float32)
        sel_total = sel_total + sel
        work = jnp.where(sel > 0, NEG_INF, work)

    w = p * sel_total
    w = w / (jnp.sum(w, axis=1, keepdims=True) + 1e-8)
    w_ref[...] = w


def _router(x, gate_w, gate_b):
    return pl.pallas_call(
        _router_body,
        out_shape=jax.ShapeDtypeStruct((N, NE), jnp.float32),
    )(x, gate_w, gate_b.reshape(1, NE))


def _dense_body(x_ref, w_ref, W1_ref, b1_ref, W2_ref, b2_ref, out_ref):
    e = pl.program_id(0)
    h = jax.lax.dot_general(
        x_ref[...], W1_ref[0], (((1,), (1,)), ((), ())),
        preferred_element_type=jnp.float32)
    h = jnp.maximum(h + b1_ref[0], 0.0)
    y = jax.lax.dot_general(
        h, W2_ref[0], (((1,), (1,)), ((), ())),
        preferred_element_type=jnp.float32)
    y = y + b2_ref[0]
    contrib = y * w_ref[0, 0, :].reshape(N, 1)

    @pl.when(e == 0)
    def _():
        out_ref[...] = contrib

    @pl.when(e > 0)
    def _():
        out_ref[...] = out_ref[...] + contrib


def _dense_moe(x, wT3, W1, b1, W2, b2):
    return pl.pallas_call(
        _dense_body,
        grid=(NE,),
        in_specs=[
            pl.BlockSpec((N, D), lambda e: (0, 0)),
            pl.BlockSpec((1, 1, N), lambda e: (e, 0, 0)),
            pl.BlockSpec((1, H, D), lambda e: (e, 0, 0)),
            pl.BlockSpec((1, H), lambda e: (e, 0)),
            pl.BlockSpec((1, D, H), lambda e: (e, 0, 0)),
            pl.BlockSpec((1, D), lambda e: (e, 0)),
        ],
        out_specs=pl.BlockSpec((N, D), lambda e: (0, 0)),
        out_shape=jax.ShapeDtypeStruct((N, D), jnp.float32),
    )(x, wT3, W1, b1, W2, b2)


@jax.jit
def kernel(x, gate_w, gate_b, W1, b1, W2, b2):
    w = _router(x, gate_w, gate_b)
    wT3 = w.T.reshape(NE, 1, N)
    return _dense_moe(x, wT3, W1, b1, W2, b2)


# speedup vs baseline: 1.0679x; 1.0679x over previous
import functools
import math

import jax
import jax.numpy as jnp
from jax.experimental import pallas as pl
from jax.experimental.pallas import tpu as pltpu

N, D, H, NE, K = 2048, 768, 768, 64, 8
TEMP_INV = 1.0 / math.e
NEG_INF = -1e30


def _router_body(x_ref, gw_ref, gb_ref, w_ref):
    scores = jax.lax.dot_general(
        x_ref[...], gw_ref[...], (((1,), (1,)), ((), ())),
        preferred_element_type=jnp.float32)
    scores = scores * TEMP_INV + gb_ref[...]
    m = jnp.max(scores, axis=1, keepdims=True)
    p = jnp.exp(scores - m)
    p = p / jnp.sum(p, axis=1, keepdims=True)

    e_iota = jax.lax.broadcasted_iota(jnp.int32, (N, NE), 1)
    work = p
    sel_total = jnp.zeros((N, NE), jnp.float32)
    for _ in range(K):
        mk = jnp.max(work, axis=1, keepdims=True)
        cand = jnp.where(work == mk, e_iota, NE)
        idx = jnp.min(cand, axis=1, keepdims=True)
        sel = (e_iota == idx).astype(jnp.float32)
        sel_total = sel_total + sel
        work = jnp.where(sel > 0, NEG_INF, work)

    w = p * sel_total
    w = w / (jnp.sum(w, axis=1, keepdims=True) + 1e-8)
    w_ref[...] = w


def _router(x, gate_w, gate_b):
    return pl.pallas_call(
        _router_body,
        out_shape=jax.ShapeDtypeStruct((N, NE), jnp.float32),
    )(x, gate_w, gate_b.reshape(1, NE))


def _dense_body(x_ref, w_ref, W1_ref, b1_ref, W2_ref, b2_ref, out_ref):
    e = pl.program_id(0)
    h = jax.lax.dot_general(
        x_ref[...], W1_ref[0], (((1,), (1,)), ((), ())),
        preferred_element_type=jnp.float32)
    h = jnp.maximum(h + b1_ref[0, 0, :], 0.0)
    y = jax.lax.dot_general(
        h, W2_ref[0], (((1,), (1,)), ((), ())),
        preferred_element_type=jnp.float32)
    y = y + b2_ref[0, 0, :]
    contrib = y * w_ref[0, 0, :].reshape(N, 1)

    @pl.when(e == 0)
    def _():
        out_ref[...] = contrib

    @pl.when(e > 0)
    def _():
        out_ref[...] = out_ref[...] + contrib


def _dense_moe(x, wT3, W1, b1, W2, b2):
    return pl.pallas_call(
        _dense_body,
        grid=(NE,),
        in_specs=[
            pl.BlockSpec((N, D), lambda e: (0, 0)),
            pl.BlockSpec((1, 1, N), lambda e: (e, 0, 0)),
            pl.BlockSpec((1, H, D), lambda e: (e, 0, 0)),
            pl.BlockSpec((1, 1, H), lambda e: (e, 0, 0)),
            pl.BlockSpec((1, D, H), lambda e: (e, 0, 0)),
            pl.BlockSpec((1, 1, D), lambda e: (e, 0, 0)),
        ],
        out_specs=pl.BlockSpec((N, D), lambda e: (0, 0)),
        out_shape=jax.ShapeDtypeStruct((N, D), jnp.float32),
    )(x, wT3, W1, b1.reshape(NE, 1, H), W2, b2.reshape(NE, 1, D))


@jax.jit
def kernel(x, gate_w, gate_b, W1, b1, W2, b2):
    w = _router(x, gate_w, gate_b)
    wT3 = w.T.reshape(NE, 1, N)
    return _dense_moe(x, wT3, W1, b1, W2, b2)
